# baseline (speedup 1.0000x reference)
import functools

import jax
import jax.numpy as jnp
from jax import lax
from jax.experimental import pallas as pl
from jax.experimental.pallas import tpu as pltpu
from jax.experimental.pallas import tpu_sc as plsc

_HIGH = jax.lax.Precision.HIGHEST
_EPS_BN = 1e-5
_EPS_D = 1e-8

MBLK = 512
SC_CHUNK = 64
SC_LANES = 16


def _prep_body(feat_ref, w1_ref, g1_ref, b1_ref, tf_ref, w2_ref, g2_ref,
               b2_ref, f_ref, sc2_ref, sh2_ref):
    B = feat_ref.shape[0]
    N = feat_ref.shape[2]
    Mtot = tf_ref.shape[2]
    w1 = w1_ref[...]
    ys = [jnp.dot(w1, feat_ref[b], preferred_element_type=jnp.float32,
                  precision=_HIGH) for b in range(B)]
    cnt1 = float(B * N)
    mean1 = sum(jnp.sum(y, axis=1, keepdims=True) for y in ys) / cnt1
    var1 = sum(jnp.sum((y - mean1) ** 2, axis=1, keepdims=True)
               for y in ys) / cnt1
    sc1 = g1_ref[...] * jax.lax.rsqrt(var1 + _EPS_BN)
    sh1 = b1_ref[...] - mean1 * sc1
    for b in range(B):
        f_ref[b] = jnp.maximum(ys[b] * sc1 + sh1, 0.0)

    cnt2 = float(B * Mtot)
    w2 = w2_ref[...]
    xmean = sum(jnp.sum(tf_ref[b], axis=1, keepdims=True)
                for b in range(B)) / cnt2
    smom = sum(jax.lax.dot_general(tf_ref[b], tf_ref[b],
                                   (((1,), (1,)), ((), ())),
                                   preferred_element_type=jnp.float32,
                                   precision=_HIGH) for b in range(B))
    mu2 = jnp.dot(w2, xmean, preferred_element_type=jnp.float32,
                  precision=_HIGH)
    ey2 = jnp.sum(jnp.dot(w2, smom, preferred_element_type=jnp.float32,
                          precision=_HIGH) * w2, axis=1,
                  keepdims=True) / cnt2
    var2 = ey2 - mu2 * mu2
    sc2 = g2_ref[...] * jax.lax.rsqrt(var2 + _EPS_BN)
    sc2_ref[...] = sc2
    sh2_ref[...] = b2_ref[...] - mu2 * sc2


def _main_body(xyz_ref, txyz_ref, tf_ref, w2_ref, sc2_ref, sh2_ref,
               pt_ref, gi0_ref, gi1_ref, gi2_ref, w0_ref, w1_ref, w2o_ref):
    S = xyz_ref[0]
    T = txyz_ref[0]
    N = S.shape[0]
    d2 = None
    for c in range(3):
        diff = S[:, c:c + 1] - T[c:c + 1, :]
        d2 = diff * diff if d2 is None else d2 + diff * diff
    iota0 = jax.lax.broadcasted_iota(jnp.int32, d2.shape, 0)
    key = jax.lax.bitcast_convert_type(
        (jax.lax.bitcast_convert_type(d2, jnp.int32) & jnp.int32(~2047))
        | iota0, jnp.float32)
    kmax = jnp.float32(jnp.inf)
    k0 = jnp.min(key, axis=0, keepdims=True)
    m1 = jnp.where(key == k0, kmax, key)
    k1 = jnp.min(m1, axis=0, keepdims=True)
    m2 = jnp.where(m1 == k1, kmax, m1)
    k2 = jnp.min(m2, axis=0, keepdims=True)
    gbase = pl.program_id(0) * N
    recips = []
    for kk, gi_ref in ((k0, gi0_ref), (k1, gi1_ref), (k2, gi2_ref)):
        kbits = jax.lax.bitcast_convert_type(kk, jnp.int32)
        gi_ref[0] = (kbits & jnp.int32(2047)) + gbase
        dq = jax.lax.bitcast_convert_type(kbits & jnp.int32(~2047),
                                          jnp.float32)
        recips.append(1.0 / (jnp.sqrt(dq) + _EPS_D))
    norm = recips[0] + recips[1] + recips[2]
    w0_ref[0] = recips[0] / norm
    w1_ref[0] = recips[1] / norm
    w2o_ref[0] = recips[2] / norm
    y2 = jnp.dot(w2_ref[...], tf_ref[0], preferred_element_type=jnp.float32,
                 precision=_HIGH)
    pt_ref[0] = jnp.maximum(y2 * sc2_ref[...] + sh2_ref[...], 0.0)


def _sc_interp_body(ftab, gi0, gi1, gi2, wh0, wh1, wh2, out,
                    ix0, ix1, ix2,
                    ga0, ga1, ga2, wa0, wa1, wa2,
                    gb0, gb1, gb2, wb0, wb1, wb2,
                    sa0, sa1, sa2, sb0, sb1, sb2, rpw, nc, c2):
    wid = lax.axis_index("s") * nc + lax.axis_index("c")
    base = wid * rpw
    nch = rpw // SC_CHUNK
    bufs = ((ga0, ga1, ga2, wa0, wa1, wa2, sa0, sa1, sa2),
            (gb0, gb1, gb2, wb0, wb1, wb2, sb0, sb1, sb2))
    pltpu.sync_copy(gi0.at[pl.ds(base, rpw)], ix0)
    pltpu.sync_copy(gi1.at[pl.ds(base, rpw)], ix1)
    pltpu.sync_copy(gi2.at[pl.ds(base, rpw)], ix2)

    def fire(ch, buf):
        r0, r1, r2, v0, v1, v2, s0, s1, s2 = bufs[buf]
        sl = pl.ds(ch * SC_CHUNK, SC_CHUNK)
        gsl = pl.ds(base + ch * SC_CHUNK, SC_CHUNK)
        pltpu.sync_copy(wh0.at[gsl], v0)
        pltpu.sync_copy(wh1.at[gsl], v1)
        pltpu.sync_copy(wh2.at[gsl], v2)
        return [pltpu.async_copy(ftab.at[ix0.at[sl]], r0, s0),
                pltpu.async_copy(ftab.at[ix1.at[sl]], r1, s1),
                pltpu.async_copy(ftab.at[ix2.at[sl]], r2, s2)]

    def consume(ch, buf, cps):
        for cp in cps:
            cp.wait()
        r0, r1, r2, v0, v1, v2, _, _, _ = bufs[buf]

        def body(t, carry):
            lane = pl.ds(0, SC_LANES)
            a0 = v0[t, lane]
            a1 = v1[t, lane]
            a2 = v2[t, lane]
            for j in range(c2 // SC_LANES):
                sl = pl.ds(j * SC_LANES, SC_LANES)
                r0[t, sl] = (a0 * r0[t, sl] + a1 * r1[t, sl]
                             + a2 * r2[t, sl])
            return carry

        lax.fori_loop(0, SC_CHUNK, body, 0)
        pltpu.sync_copy(r0, out.at[pl.ds(base + ch * SC_CHUNK, SC_CHUNK)])

    prev = fire(0, 0)
    for ch in range(1, nch):
        cur = fire(ch, ch % 2)
        consume(ch - 1, (ch - 1) % 2, prev)
        prev = cur
    consume(nch - 1, (nch - 1) % 2, prev)


@jax.jit
def kernel(xyz, feature, target_xyz, target_feature, W1, gamma1, beta1, W2,
           gamma2, beta2):
    B, N, _ = xyz.shape
    M = target_xyz.shape[1]
    C2 = W1.shape[0]
    BM = B * M
    txyz_t = jnp.transpose(target_xyz, (0, 2, 1))
    g1 = gamma1.reshape(C2, 1)
    b1 = beta1.reshape(C2, 1)
    g2 = gamma2.reshape(C2, 1)
    b2 = beta2.reshape(C2, 1)

    f, sc2, sh2 = pl.pallas_call(
        _prep_body,
        out_shape=[
            jax.ShapeDtypeStruct((B, C2, N), jnp.float32),
            jax.ShapeDtypeStruct((C2, 1), jnp.float32),
            jax.ShapeDtypeStruct((C2, 1), jnp.float32),
        ],
    )(feature, W1, g1, b1, target_feature, W2, g2, b2)

    grid = (B, M // MBLK)
    blk_row_i = jax.ShapeDtypeStruct((B, 1, M), jnp.int32)
    blk_row_f = jax.ShapeDtypeStruct((B, 1, M), jnp.float32)
    row_spec = pl.BlockSpec((1, 1, MBLK), lambda b, j: (b, 0, j))
    part, gi0, gi1, gi2, w0, w1, w2 = pl.pallas_call(
        _main_body,
        grid=grid,
        in_specs=[
            pl.BlockSpec((1, N, 3), lambda b, j: (b, 0, 0)),
            pl.BlockSpec((1, 3, MBLK), lambda b, j: (b, 0, j)),
            pl.BlockSpec((1, C2, MBLK), lambda b, j: (b, 0, j)),
            pl.BlockSpec((C2, C2), lambda b, j: (0, 0)),
            pl.BlockSpec((C2, 1), lambda b, j: (0, 0)),
            pl.BlockSpec((C2, 1), lambda b, j: (0, 0)),
        ],
        out_specs=[
            pl.BlockSpec((1, C2, MBLK), lambda b, j: (b, 0, j)),
            row_spec, row_spec, row_spec, row_spec, row_spec, row_spec,
        ],
        out_shape=[
            jax.ShapeDtypeStruct((B, C2, M), jnp.float32),
            blk_row_i, blk_row_i, blk_row_i,
            blk_row_f, blk_row_f, blk_row_f,
        ],
    )(xyz, txyz_t, target_feature, W2, sc2, sh2)

    ftab = jnp.transpose(f, (0, 2, 1)).reshape(B * N, C2)
    info = plsc.get_sparse_core_info()
    nw = info.num_cores * info.num_subcores
    rpw = BM // nw
    mesh = plsc.VectorSubcoreMesh(core_axis_name="c", subcore_axis_name="s")
    interp_t = pl.kernel(
        functools.partial(_sc_interp_body, rpw=rpw, nc=info.num_cores, c2=C2),
        mesh=mesh,
        out_type=jax.ShapeDtypeStruct((BM, C2), jnp.float32),
        scratch_types=(
            [pltpu.VMEM((rpw,), jnp.int32)] * 3
            + ([pltpu.VMEM((SC_CHUNK, C2), jnp.float32)] * 3
               + [pltpu.VMEM((SC_CHUNK, SC_LANES), jnp.float32)] * 3) * 2
            + [pltpu.SemaphoreType.DMA] * 6
        ),
    )(ftab, gi0.reshape(BM), gi1.reshape(BM), gi2.reshape(BM),
      jnp.broadcast_to(w0.reshape(BM, 1), (BM, SC_LANES)),
      jnp.broadcast_to(w1.reshape(BM, 1), (BM, SC_LANES)),
      jnp.broadcast_to(w2.reshape(BM, 1), (BM, SC_LANES)))

    out = part + jnp.transpose(interp_t.reshape(B, M, C2), (0, 2, 1))
    return out


# speedup vs baseline: 1.0432x; 1.0432x over previous
import functools

import jax
import jax.numpy as jnp
from jax import lax
from jax.experimental import pallas as pl
from jax.experimental.pallas import tpu as pltpu
from jax.experimental.pallas import tpu_sc as plsc

_HIGH = jax.lax.Precision.HIGHEST
_EPS_BN = 1e-5
_EPS_D = 1e-8

MBLK = 512
CBLK = 2048
SC_CHUNK = 128
SC_LANES = 16


def _prep_body(feat_ref, w1_ref, g1_ref, b1_ref, tf_ref, w2_ref, g2_ref,
               b2_ref, f_ref, sc2_ref, sh2_ref):
    B = feat_ref.shape[0]
    N = feat_ref.shape[2]
    Mtot = tf_ref.shape[2]
    w1 = w1_ref[...]
    ys = [jnp.dot(w1, feat_ref[b], preferred_element_type=jnp.float32,
                  precision=_HIGH) for b in range(B)]
    cnt1 = float(B * N)
    mean1 = sum(jnp.sum(y, axis=1, keepdims=True) for y in ys) / cnt1
    var1 = sum(jnp.sum((y - mean1) ** 2, axis=1, keepdims=True)
               for y in ys) / cnt1
    sc1 = g1_ref[...] * jax.lax.rsqrt(var1 + _EPS_BN)
    sh1 = b1_ref[...] - mean1 * sc1
    for b in range(B):
        f_ref[b] = jnp.maximum(ys[b] * sc1 + sh1, 0.0)

    cnt2 = float(B * Mtot)
    w2 = w2_ref[...]
    xmean = sum(jnp.sum(tf_ref[b], axis=1, keepdims=True)
                for b in range(B)) / cnt2
    smom = sum(jax.lax.dot_general(tf_ref[b], tf_ref[b],
                                   (((1,), (1,)), ((), ())),
                                   preferred_element_type=jnp.float32,
                                   precision=_HIGH) for b in range(B))
    mu2 = jnp.dot(w2, xmean, preferred_element_type=jnp.float32,
                  precision=_HIGH)
    ey2 = jnp.sum(jnp.dot(w2, smom, preferred_element_type=jnp.float32,
                          precision=_HIGH) * w2, axis=1,
                  keepdims=True) / cnt2
    var2 = ey2 - mu2 * mu2
    sc2 = g2_ref[...] * jax.lax.rsqrt(var2 + _EPS_BN)
    sc2_ref[...] = sc2
    sh2_ref[...] = b2_ref[...] - mu2 * sc2


def _main_body(xyz_ref, txyz_ref, tf_ref, w2_ref, sc2_ref, sh2_ref,
               pt_ref, gi0_ref, gi1_ref, gi2_ref, w0_ref, w1_ref, w2o_ref):
    S = xyz_ref[0]
    T = txyz_ref[0]
    N = S.shape[0]
    d2 = None
    for c in range(3):
        diff = S[:, c:c + 1] - T[c:c + 1, :]
        d2 = diff * diff if d2 is None else d2 + diff * diff
    iota0 = jax.lax.broadcasted_iota(jnp.int32, d2.shape, 0)
    key = jax.lax.bitcast_convert_type(
        (jax.lax.bitcast_convert_type(d2, jnp.int32) & jnp.int32(~2047))
        | iota0, jnp.float32)
    kmax = jnp.float32(jnp.inf)
    k0 = jnp.min(key, axis=0, keepdims=True)
    m1 = jnp.where(key == k0, kmax, key)
    k1 = jnp.min(m1, axis=0, keepdims=True)
    m2 = jnp.where(m1 == k1, kmax, m1)
    k2 = jnp.min(m2, axis=0, keepdims=True)
    gbase = pl.program_id(0) * N
    recips = []
    for kk, gi_ref in ((k0, gi0_ref), (k1, gi1_ref), (k2, gi2_ref)):
        kbits = jax.lax.bitcast_convert_type(kk, jnp.int32)
        gi_ref[0] = (kbits & jnp.int32(2047)) + gbase
        dq = jax.lax.bitcast_convert_type(kbits & jnp.int32(~2047),
                                          jnp.float32)
        recips.append(1.0 / (jnp.sqrt(dq) + _EPS_D))
    norm = recips[0] + recips[1] + recips[2]
    w0_ref[0] = recips[0] / norm
    w1_ref[0] = recips[1] / norm
    w2o_ref[0] = recips[2] / norm
    y2 = jnp.dot(w2_ref[...], tf_ref[0], preferred_element_type=jnp.float32,
                 precision=_HIGH)
    pt_ref[0] = jnp.maximum(y2 * sc2_ref[...] + sh2_ref[...], 0.0)


def _sc_gather_body(ftab, gi0, gi1, gi2, r0o, r1o, r2o,
                    ix0, ix1, ix2,
                    ga0, ga1, ga2, gb0, gb1, gb2,
                    sa0, sa1, sa2, sb0, sb1, sb2, rpw, nc):
    wid = lax.axis_index("s") * nc + lax.axis_index("c")
    base = wid * rpw
    nch = rpw // SC_CHUNK
    bufs = ((ga0, ga1, ga2, sa0, sa1, sa2),
            (gb0, gb1, gb2, sb0, sb1, sb2))
    pltpu.sync_copy(gi0.at[pl.ds(base, rpw)], ix0)
    pltpu.sync_copy(gi1.at[pl.ds(base, rpw)], ix1)
    pltpu.sync_copy(gi2.at[pl.ds(base, rpw)], ix2)

    def fire(ch, buf):
        r0, r1, r2, s0, s1, s2 = bufs[buf]
        sl = pl.ds(ch * SC_CHUNK, SC_CHUNK)
        return [pltpu.async_copy(ftab.at[ix0.at[sl]], r0, s0),
                pltpu.async_copy(ftab.at[ix1.at[sl]], r1, s1),
                pltpu.async_copy(ftab.at[ix2.at[sl]], r2, s2)]

    def consume(ch, buf, cps):
        for cp in cps:
            cp.wait()
        r0, r1, r2 = bufs[buf][:3]
        gsl = pl.ds(base + ch * SC_CHUNK, SC_CHUNK)
        pltpu.sync_copy(r0, r0o.at[gsl])
        pltpu.sync_copy(r1, r1o.at[gsl])
        pltpu.sync_copy(r2, r2o.at[gsl])

    prev = fire(0, 0)
    for ch in range(1, nch):
        cur = fire(ch, ch % 2)
        consume(ch - 1, (ch - 1) % 2, prev)
        prev = cur
    consume(nch - 1, (nch - 1) % 2, prev)


def _combine_body(r0_ref, r1_ref, r2_ref, w0_ref, w1_ref, w2_ref, o_ref):
    o_ref[0] = (w0_ref[0] * r0_ref[0].astype(jnp.float32)
                + w1_ref[0] * r1_ref[0].astype(jnp.float32)
                + w2_ref[0] * r2_ref[0].astype(jnp.float32))


@jax.jit
def kernel(xyz, feature, target_xyz, target_feature, W1, gamma1, beta1, W2,
           gamma2, beta2):
    B, N, _ = xyz.shape
    M = target_xyz.shape[1]
    C2 = W1.shape[0]
    BM = B * M
    txyz_t = jnp.transpose(target_xyz, (0, 2, 1))
    g1 = gamma1.reshape(C2, 1)
    b1 = beta1.reshape(C2, 1)
    g2 = gamma2.reshape(C2, 1)
    b2 = beta2.reshape(C2, 1)

    f, sc2, sh2 = pl.pallas_call(
        _prep_body,
        out_shape=[
            jax.ShapeDtypeStruct((B, C2, N), jnp.float32),
            jax.ShapeDtypeStruct((C2, 1), jnp.float32),
            jax.ShapeDtypeStruct((C2, 1), jnp.float32),
        ],
    )(feature, W1, g1, b1, target_feature, W2, g2, b2)

    grid = (B, M // MBLK)
    blk_row_i = jax.ShapeDtypeStruct((B, 1, M), jnp.int32)
    blk_row_f = jax.ShapeDtypeStruct((B, 1, M), jnp.float32)
    row_spec = pl.BlockSpec((1, 1, MBLK), lambda b, j: (b, 0, j))
    part, gi0, gi1, gi2, w0, w1, w2 = pl.pallas_call(
        _main_body,
        grid=grid,
        in_specs=[
            pl.BlockSpec((1, N, 3), lambda b, j: (b, 0, 0)),
            pl.BlockSpec((1, 3, MBLK), lambda b, j: (b, 0, j)),
            pl.BlockSpec((1, C2, MBLK), lambda b, j: (b, 0, j)),
            pl.BlockSpec((C2, C2), lambda b, j: (0, 0)),
            pl.BlockSpec((C2, 1), lambda b, j: (0, 0)),
            pl.BlockSpec((C2, 1), lambda b, j: (0, 0)),
        ],
        out_specs=[
            pl.BlockSpec((1, C2, MBLK), lambda b, j: (b, 0, j)),
            row_spec, row_spec, row_spec, row_spec, row_spec, row_spec,
        ],
        out_shape=[
            jax.ShapeDtypeStruct((B, C2, M), jnp.float32),
            blk_row_i, blk_row_i, blk_row_i,
            blk_row_f, blk_row_f, blk_row_f,
        ],
    )(xyz, txyz_t, target_feature, W2, sc2, sh2)

    ftab = jnp.transpose(f, (0, 2, 1)).reshape(B * N, C2)
    info = plsc.get_sparse_core_info()
    nw = info.num_cores * info.num_subcores
    rpw = BM // nw
    mesh = plsc.VectorSubcoreMesh(core_axis_name="c", subcore_axis_name="s")
    rows_ty = jax.ShapeDtypeStruct((BM, C2), jnp.float32)
    r0, r1, r2 = pl.kernel(
        functools.partial(_sc_gather_body, rpw=rpw, nc=info.num_cores),
        mesh=mesh,
        out_type=[rows_ty, rows_ty, rows_ty],
        scratch_types=(
            [pltpu.VMEM((rpw,), jnp.int32)] * 3
            + [pltpu.VMEM((SC_CHUNK, C2), jnp.float32)] * 6
            + [pltpu.SemaphoreType.DMA] * 6
        ),
    )(ftab, gi0.reshape(BM), gi1.reshape(BM), gi2.reshape(BM))
    r0, r1, r2 = (r.reshape(B, M, C2) for r in (r0, r1, r2))

    cgrid = (B, M // CBLK)
    blk_spec = pl.BlockSpec((1, CBLK, C2), lambda b, j: (b, j, 0))
    col_spec = pl.BlockSpec((1, CBLK, 1), lambda b, j: (b, j, 0))
    interp_t = pl.pallas_call(
        _combine_body,
        grid=cgrid,
        in_specs=[blk_spec, blk_spec, blk_spec,
                  col_spec, col_spec, col_spec],
        out_specs=blk_spec,
        out_shape=jax.ShapeDtypeStruct((B, M, C2), jnp.float32),
    )(r0, r1, r2,
      jnp.transpose(w0, (0, 2, 1)), jnp.transpose(w1, (0, 2, 1)),
      jnp.transpose(w2, (0, 2, 1)))

    out = part + jnp.transpose(interp_t, (0, 2, 1))
    return out


# speedup vs baseline: 1.0550x; 1.0113x over previous
import functools

import jax
import jax.numpy as jnp
from jax import lax
from jax.experimental import pallas as pl
from jax.experimental.pallas import tpu as pltpu
from jax.experimental.pallas import tpu_sc as plsc

_HIGH = jax.lax.Precision.HIGHEST
_EPS_BN = 1e-5
_EPS_D = 1e-8

MBLK = 1024
CBLK = 2048
SC_CHUNK = 128
SC_LANES = 16


def _prep_body(feat_ref, w1_ref, g1_ref, b1_ref, tf_ref, w2_ref, g2_ref,
               b2_ref, f_ref, sc2_ref, sh2_ref):
    B = feat_ref.shape[0]
    N = feat_ref.shape[2]
    Mtot = tf_ref.shape[2]
    w1 = w1_ref[...]
    ys = [jnp.dot(w1, feat_ref[b], preferred_element_type=jnp.float32,
                  precision=_HIGH) for b in range(B)]
    cnt1 = float(B * N)
    mean1 = sum(jnp.sum(y, axis=1, keepdims=True) for y in ys) / cnt1
    var1 = sum(jnp.sum((y - mean1) ** 2, axis=1, keepdims=True)
               for y in ys) / cnt1
    sc1 = g1_ref[...] * jax.lax.rsqrt(var1 + _EPS_BN)
    sh1 = b1_ref[...] - mean1 * sc1
    for b in range(B):
        f_ref[b] = jnp.maximum(ys[b] * sc1 + sh1, 0.0)

    cnt2 = float(B * Mtot)
    w2 = w2_ref[...]
    xmean = sum(jnp.sum(tf_ref[b], axis=1, keepdims=True)
                for b in range(B)) / cnt2
    smom = sum(jax.lax.dot_general(tf_ref[b], tf_ref[b],
                                   (((1,), (1,)), ((), ())),
                                   preferred_element_type=jnp.float32,
                                   precision=_HIGH) for b in range(B))
    mu2 = jnp.dot(w2, xmean, preferred_element_type=jnp.float32,
                  precision=_HIGH)
    ey2 = jnp.sum(jnp.dot(w2, smom, preferred_element_type=jnp.float32,
                          precision=_HIGH) * w2, axis=1,
                  keepdims=True) / cnt2
    var2 = ey2 - mu2 * mu2
    sc2 = g2_ref[...] * jax.lax.rsqrt(var2 + _EPS_BN)
    sc2_ref[...] = sc2
    sh2_ref[...] = b2_ref[...] - mu2 * sc2


def _main_body(xyz_ref, txyz_ref, tf_ref, w2_ref, sc2_ref, sh2_ref,
               pt_ref, gi0_ref, gi1_ref, gi2_ref, w0_ref, w1_ref, w2o_ref):
    S = xyz_ref[0]
    T = txyz_ref[0]
    N = S.shape[0]
    d2 = None
    for c in range(3):
        diff = S[:, c:c + 1] - T[c:c + 1, :]
        d2 = diff * diff if d2 is None else d2 + diff * diff
    iota0 = jax.lax.broadcasted_iota(jnp.int32, d2.shape, 0)
    key = jax.lax.bitcast_convert_type(
        (jax.lax.bitcast_convert_type(d2, jnp.int32) & jnp.int32(~2047))
        | iota0, jnp.float32)
    kmax = jnp.float32(jnp.inf)
    k0 = jnp.min(key, axis=0, keepdims=True)
    m1 = jnp.where(key == k0, kmax, key)
    k1 = jnp.min(m1, axis=0, keepdims=True)
    m2 = jnp.where(m1 == k1, kmax, m1)
    k2 = jnp.min(m2, axis=0, keepdims=True)
    gbase = pl.program_id(0) * N
    recips = []
    for kk, gi_ref in ((k0, gi0_ref), (k1, gi1_ref), (k2, gi2_ref)):
        kbits = jax.lax.bitcast_convert_type(kk, jnp.int32)
        gi_ref[0] = (kbits & jnp.int32(2047)) + gbase
        dq = jax.lax.bitcast_convert_type(kbits & jnp.int32(~2047),
                                          jnp.float32)
        recips.append(1.0 / (jnp.sqrt(dq) + _EPS_D))
    norm = recips[0] + recips[1] + recips[2]
    w0_ref[0] = recips[0] / norm
    w1_ref[0] = recips[1] / norm
    w2o_ref[0] = recips[2] / norm
    y2 = jnp.dot(w2_ref[...], tf_ref[0], preferred_element_type=jnp.float32,
                 precision=_HIGH)
    pt_ref[0] = jnp.maximum(y2 * sc2_ref[...] + sh2_ref[...], 0.0)


def _sc_gather_body(ftab, gi0, gi1, gi2, r0o, r1o, r2o,
                    ix0, ix1, ix2,
                    ga0, ga1, ga2, gb0, gb1, gb2,
                    sa0, sa1, sa2, sb0, sb1, sb2, rpw, nc):
    wid = lax.axis_index("s") * nc + lax.axis_index("c")
    base = wid * rpw
    nch = rpw // SC_CHUNK
    bufs = ((ga0, ga1, ga2, sa0, sa1, sa2),
            (gb0, gb1, gb2, sb0, sb1, sb2))
    pltpu.sync_copy(gi0.at[pl.ds(base, rpw)], ix0)
    pltpu.sync_copy(gi1.at[pl.ds(base, rpw)], ix1)
    pltpu.sync_copy(gi2.at[pl.ds(base, rpw)], ix2)

    def fire(ch, buf):
        r0, r1, r2, s0, s1, s2 = bufs[buf]
        sl = pl.ds(ch * SC_CHUNK, SC_CHUNK)
        return [pltpu.async_copy(ftab.at[ix0.at[sl]], r0, s0),
                pltpu.async_copy(ftab.at[ix1.at[sl]], r1, s1),
                pltpu.async_copy(ftab.at[ix2.at[sl]], r2, s2)]

    def consume(ch, buf, cps):
        for cp in cps:
            cp.wait()
        r0, r1, r2 = bufs[buf][:3]
        gsl = pl.ds(base + ch * SC_CHUNK, SC_CHUNK)
        pltpu.sync_copy(r0, r0o.at[gsl])
        pltpu.sync_copy(r1, r1o.at[gsl])
        pltpu.sync_copy(r2, r2o.at[gsl])

    prev = fire(0, 0)
    for ch in range(1, nch):
        cur = fire(ch, ch % 2)
        consume(ch - 1, (ch - 1) % 2, prev)
        prev = cur
    consume(nch - 1, (nch - 1) % 2, prev)


def _combine_body(r0_ref, r1_ref, r2_ref, w0_ref, w1_ref, w2_ref, o_ref):
    o_ref[0] = (w0_ref[0] * r0_ref[0].astype(jnp.float32)
                + w1_ref[0] * r1_ref[0].astype(jnp.float32)
                + w2_ref[0] * r2_ref[0].astype(jnp.float32))


@jax.jit
def kernel(xyz, feature, target_xyz, target_feature, W1, gamma1, beta1, W2,
           gamma2, beta2):
    B, N, _ = xyz.shape
    M = target_xyz.shape[1]
    C2 = W1.shape[0]
    BM = B * M
    txyz_t = jnp.transpose(target_xyz, (0, 2, 1))
    g1 = gamma1.reshape(C2, 1)
    b1 = beta1.reshape(C2, 1)
    g2 = gamma2.reshape(C2, 1)
    b2 = beta2.reshape(C2, 1)

    f, sc2, sh2 = pl.pallas_call(
        _prep_body,
        out_shape=[
            jax.ShapeDtypeStruct((B, C2, N), jnp.float32),
            jax.ShapeDtypeStruct((C2, 1), jnp.float32),
            jax.ShapeDtypeStruct((C2, 1), jnp.float32),
        ],
    )(feature, W1, g1, b1, target_feature, W2, g2, b2)

    grid = (B, M // MBLK)
    blk_row_i = jax.ShapeDtypeStruct((B, 1, M), jnp.int32)
    blk_row_f = jax.ShapeDtypeStruct((B, 1, M), jnp.float32)
    row_spec = pl.BlockSpec((1, 1, MBLK), lambda b, j: (b, 0, j))
    part, gi0, gi1, gi2, w0, w1, w2 = pl.pallas_call(
        _main_body,
        grid=grid,
        in_specs=[
            pl.BlockSpec((1, N, 3), lambda b, j: (b, 0, 0)),
            pl.BlockSpec((1, 3, MBLK), lambda b, j: (b, 0, j)),
            pl.BlockSpec((1, C2, MBLK), lambda b, j: (b, 0, j)),
            pl.BlockSpec((C2, C2), lambda b, j: (0, 0)),
            pl.BlockSpec((C2, 1), lambda b, j: (0, 0)),
            pl.BlockSpec((C2, 1), lambda b, j: (0, 0)),
        ],
        out_specs=[
            pl.BlockSpec((1, C2, MBLK), lambda b, j: (b, 0, j)),
            row_spec, row_spec, row_spec, row_spec, row_spec, row_spec,
        ],
        out_shape=[
            jax.ShapeDtypeStruct((B, C2, M), jnp.float32),
            blk_row_i, blk_row_i, blk_row_i,
            blk_row_f, blk_row_f, blk_row_f,
        ],
    )(xyz, txyz_t, target_feature, W2, sc2, sh2)

    ftab = jnp.transpose(f, (0, 2, 1)).reshape(B * N, C2)
    info = plsc.get_sparse_core_info()
    nw = info.num_cores * info.num_subcores
    rpw = BM // nw
    mesh = plsc.VectorSubcoreMesh(core_axis_name="c", subcore_axis_name="s")
    rows_ty = jax.ShapeDtypeStruct((BM, C2), jnp.float32)
    r0, r1, r2 = pl.kernel(
        functools.partial(_sc_gather_body, rpw=rpw, nc=info.num_cores),
        mesh=mesh,
        out_type=[rows_ty, rows_ty, rows_ty],
        scratch_types=(
            [pltpu.VMEM((rpw,), jnp.int32)] * 3
            + [pltpu.VMEM((SC_CHUNK, C2), jnp.float32)] * 6
            + [pltpu.SemaphoreType.DMA] * 6
        ),
    )(ftab, gi0.reshape(BM), gi1.reshape(BM), gi2.reshape(BM))
    r0, r1, r2 = (r.reshape(B, M, C2) for r in (r0, r1, r2))

    cgrid = (B, M // CBLK)
    blk_spec = pl.BlockSpec((1, CBLK, C2), lambda b, j: (b, j, 0))
    col_spec = pl.BlockSpec((1, CBLK, 1), lambda b, j: (b, j, 0))
    interp_t = pl.pallas_call(
        _combine_body,
        grid=cgrid,
        in_specs=[blk_spec, blk_spec, blk_spec,
                  col_spec, col_spec, col_spec],
        out_specs=blk_spec,
        out_shape=jax.ShapeDtypeStruct((B, M, C2), jnp.float32),
    )(r0, r1, r2,
      jnp.transpose(w0, (0, 2, 1)), jnp.transpose(w1, (0, 2, 1)),
      jnp.transpose(w2, (0, 2, 1)))

    out = part + jnp.transpose(interp_t, (0, 2, 1))
    return out


# speedup vs baseline: 1.0551x; 1.0001x over previous
import functools

import jax
import jax.numpy as jnp
from jax import lax
from jax.experimental import pallas as pl
from jax.experimental.pallas import tpu as pltpu
from jax.experimental.pallas import tpu_sc as plsc

_HIGH = jax.lax.Precision.HIGHEST
_EPS_BN = 1e-5
_EPS_D = 1e-8

MBLK = 1024
CBLK = 2048
SC_CHUNK = 128
SC_LANES = 16


def _prep_body(feat_ref, w1_ref, g1_ref, b1_ref, tf_ref, w2_ref, g2_ref,
               b2_ref, f_ref, sc2_ref, sh2_ref):
    B = feat_ref.shape[0]
    N = feat_ref.shape[2]
    Mtot = tf_ref.shape[2]
    w1 = w1_ref[...]
    ys = [jnp.dot(w1, feat_ref[b], preferred_element_type=jnp.float32,
                  precision=_HIGH) for b in range(B)]
    cnt1 = float(B * N)
    mean1 = sum(jnp.sum(y, axis=1, keepdims=True) for y in ys) / cnt1
    var1 = sum(jnp.sum((y - mean1) ** 2, axis=1, keepdims=True)
               for y in ys) / cnt1
    sc1 = g1_ref[...] * jax.lax.rsqrt(var1 + _EPS_BN)
    sh1 = b1_ref[...] - mean1 * sc1
    for b in range(B):
        f_ref[b] = jnp.maximum(ys[b] * sc1 + sh1, 0.0)

    cnt2 = float(B * Mtot)
    w2 = w2_ref[...]
    xmean = sum(jnp.sum(tf_ref[b], axis=1, keepdims=True)
                for b in range(B)) / cnt2
    smom = sum(jax.lax.dot_general(tf_ref[b], tf_ref[b],
                                   (((1,), (1,)), ((), ())),
                                   preferred_element_type=jnp.float32,
                                   precision=_HIGH) for b in range(B))
    mu2 = jnp.dot(w2, xmean, preferred_element_type=jnp.float32,
                  precision=_HIGH)
    ey2 = jnp.sum(jnp.dot(w2, smom, preferred_element_type=jnp.float32,
                          precision=_HIGH) * w2, axis=1,
                  keepdims=True) / cnt2
    var2 = ey2 - mu2 * mu2
    sc2 = g2_ref[...] * jax.lax.rsqrt(var2 + _EPS_BN)
    sc2_ref[...] = sc2
    sh2_ref[...] = b2_ref[...] - mu2 * sc2


def _main_body(xyz_ref, txyz_ref, tf_ref, w2_ref, sc2_ref, sh2_ref,
               pt_ref, gi0_ref, gi1_ref, gi2_ref, w0_ref, w1_ref, w2o_ref):
    S = xyz_ref[0]
    T = txyz_ref[0]
    N = S.shape[0]
    d2 = None
    for c in range(3):
        diff = S[:, c:c + 1] - T[c:c + 1, :]
        d2 = diff * diff if d2 is None else d2 + diff * diff
    iota0 = jax.lax.broadcasted_iota(jnp.int32, d2.shape, 0)
    key = jax.lax.bitcast_convert_type(
        (jax.lax.bitcast_convert_type(d2, jnp.int32) & jnp.int32(~2047))
        | iota0, jnp.float32)
    kmax = jnp.float32(jnp.inf)
    k0 = jnp.min(key, axis=0, keepdims=True)
    m1 = jnp.where(key == k0, kmax, key)
    k1 = jnp.min(m1, axis=0, keepdims=True)
    m2 = jnp.where(m1 == k1, kmax, m1)
    k2 = jnp.min(m2, axis=0, keepdims=True)
    gbase = pl.program_id(0) * N
    recips = []
    for kk, gi_ref in ((k0, gi0_ref), (k1, gi1_ref), (k2, gi2_ref)):
        kbits = jax.lax.bitcast_convert_type(kk, jnp.int32)
        gi_ref[0] = (kbits & jnp.int32(2047)) + gbase
        dq = jax.lax.bitcast_convert_type(kbits & jnp.int32(~2047),
                                          jnp.float32)
        recips.append(1.0 / (jnp.sqrt(dq) + _EPS_D))
    norm = recips[0] + recips[1] + recips[2]
    w0_ref[0] = recips[0] / norm
    w1_ref[0] = recips[1] / norm
    w2o_ref[0] = recips[2] / norm
    y2 = jnp.dot(w2_ref[...], tf_ref[0], preferred_element_type=jnp.float32,
                 precision=_HIGH)
    pt_ref[0] = jnp.maximum(y2 * sc2_ref[...] + sh2_ref[...], 0.0)


def _sc_gather_body(ftab, gi0, gi1, gi2, r0o, r1o, r2o,
                    ix0, ix1, ix2,
                    ga0, ga1, ga2, gb0, gb1, gb2,
                    sa0, sa1, sa2, sb0, sb1, sb2, rpw, nc):
    wid = lax.axis_index("s") * nc + lax.axis_index("c")
    base = wid * rpw
    nch = rpw // SC_CHUNK
    bufs = ((ga0, ga1, ga2, sa0, sa1, sa2),
            (gb0, gb1, gb2, sb0, sb1, sb2))
    pltpu.sync_copy(gi0.at[pl.ds(base, rpw)], ix0)
    pltpu.sync_copy(gi1.at[pl.ds(base, rpw)], ix1)
    pltpu.sync_copy(gi2.at[pl.ds(base, rpw)], ix2)

    def fire(ch, buf):
        r0, r1, r2, s0, s1, s2 = bufs[buf]
        sl = pl.ds(ch * SC_CHUNK, SC_CHUNK)
        return [pltpu.async_copy(ftab.at[ix0.at[sl]], r0, s0),
                pltpu.async_copy(ftab.at[ix1.at[sl]], r1, s1),
                pltpu.async_copy(ftab.at[ix2.at[sl]], r2, s2)]

    def consume(ch, buf, cps):
        for cp in cps:
            cp.wait()
        r0, r1, r2 = bufs[buf][:3]
        gsl = pl.ds(base + ch * SC_CHUNK, SC_CHUNK)
        pltpu.sync_copy(r0, r0o.at[gsl])
        pltpu.sync_copy(r1, r1o.at[gsl])
        pltpu.sync_copy(r2, r2o.at[gsl])

    prev = fire(0, 0)
    for ch in range(1, nch):
        cur = fire(ch, ch % 2)
        consume(ch - 1, (ch - 1) % 2, prev)
        prev = cur
    consume(nch - 1, (nch - 1) % 2, prev)


def _combine_body(r0_ref, r1_ref, r2_ref, w0_ref, w1_ref, w2_ref, o_ref):
    o_ref[0] = (w0_ref[0] * r0_ref[0]
                + w1_ref[0] * r1_ref[0]
                + w2_ref[0] * r2_ref[0])


@jax.jit
def kernel(xyz, feature, target_xyz, target_feature, W1, gamma1, beta1, W2,
           gamma2, beta2):
    B, N, _ = xyz.shape
    M = target_xyz.shape[1]
    C2 = W1.shape[0]
    BM = B * M
    txyz_t = jnp.transpose(target_xyz, (0, 2, 1))
    g1 = gamma1.reshape(C2, 1)
    b1 = beta1.reshape(C2, 1)
    g2 = gamma2.reshape(C2, 1)
    b2 = beta2.reshape(C2, 1)

    f, sc2, sh2 = pl.pallas_call(
        _prep_body,
        out_shape=[
            jax.ShapeDtypeStruct((B, C2, N), jnp.float32),
            jax.ShapeDtypeStruct((C2, 1), jnp.float32),
            jax.ShapeDtypeStruct((C2, 1), jnp.float32),
        ],
    )(feature, W1, g1, b1, target_feature, W2, g2, b2)

    grid = (B, M // MBLK)
    blk_row_i = jax.ShapeDtypeStruct((B, 1, M), jnp.int32)
    blk_row_f = jax.ShapeDtypeStruct((B, 1, M), jnp.float32)
    row_spec = pl.BlockSpec((1, 1, MBLK), lambda b, j: (b, 0, j))
    part, gi0, gi1, gi2, w0, w1, w2 = pl.pallas_call(
        _main_body,
        grid=grid,
        in_specs=[
            pl.BlockSpec((1, N, 3), lambda b, j: (b, 0, 0)),
            pl.BlockSpec((1, 3, MBLK), lambda b, j: (b, 0, j)),
            pl.BlockSpec((1, C2, MBLK), lambda b, j: (b, 0, j)),
            pl.BlockSpec((C2, C2), lambda b, j: (0, 0)),
            pl.BlockSpec((C2, 1), lambda b, j: (0, 0)),
            pl.BlockSpec((C2, 1), lambda b, j: (0, 0)),
        ],
        out_specs=[
            pl.BlockSpec((1, C2, MBLK), lambda b, j: (b, 0, j)),
            row_spec, row_spec, row_spec, row_spec, row_spec, row_spec,
        ],
        out_shape=[
            jax.ShapeDtypeStruct((B, C2, M), jnp.float32),
            blk_row_i, blk_row_i, blk_row_i,
            blk_row_f, blk_row_f, blk_row_f,
        ],
    )(xyz, txyz_t, target_feature, W2, sc2, sh2)

    ftab = jnp.transpose(f, (0, 2, 1)).reshape(B * N, C2)
    info = plsc.get_sparse_core_info()
    nw = info.num_cores * info.num_subcores
    rpw = BM // nw
    mesh = plsc.VectorSubcoreMesh(core_axis_name="c", subcore_axis_name="s")
    rows_ty = jax.ShapeDtypeStruct((BM, C2), jnp.float32)
    r0, r1, r2 = pl.kernel(
        functools.partial(_sc_gather_body, rpw=rpw, nc=info.num_cores),
        mesh=mesh,
        out_type=[rows_ty, rows_ty, rows_ty],
        scratch_types=(
            [pltpu.VMEM((rpw,), jnp.int32)] * 3
            + [pltpu.VMEM((SC_CHUNK, C2), jnp.float32)] * 6
            + [pltpu.SemaphoreType.DMA] * 6
        ),
    )(ftab, gi0.reshape(BM), gi1.reshape(BM), gi2.reshape(BM))
    r0, r1, r2 = (r.reshape(B, M, C2) for r in (r0, r1, r2))

    cgrid = (B, M // CBLK)
    blk_spec = pl.BlockSpec((1, CBLK, C2), lambda b, j: (b, j, 0))
    col_spec = pl.BlockSpec((1, CBLK, 1), lambda b, j: (b, j, 0))
    interp_t = pl.pallas_call(
        _combine_body,
        grid=cgrid,
        in_specs=[blk_spec, blk_spec, blk_spec,
                  col_spec, col_spec, col_spec],
        out_specs=blk_spec,
        out_shape=jax.ShapeDtypeStruct((B, M, C2), jnp.float32),
    )(r0, r1, r2,
      jnp.transpose(w0, (0, 2, 1)), jnp.transpose(w1, (0, 2, 1)),
      jnp.transpose(w2, (0, 2, 1)))

    out = part + jnp.transpose(interp_t, (0, 2, 1))
    return out


# speedup vs baseline: 1.1788x; 1.1173x over previous
import functools

import jax
import jax.numpy as jnp
from jax import lax
from jax.experimental import pallas as pl
from jax.experimental.pallas import tpu as pltpu
from jax.experimental.pallas import tpu_sc as plsc

_HIGH = jax.lax.Precision.HIGHEST
_EPS_BN = 1e-5
_EPS_D = 1e-8

MBLK = 1024
CBLK = 2048
SC_CHUNK = 128


def _prep_body(feat_ref, w1_ref, g1_ref, b1_ref, tf_ref, w2_ref, g2_ref,
               b2_ref, f_ref, sc2_ref, sh2_ref):
    B = feat_ref.shape[0]
    N = feat_ref.shape[2]
    Mtot = tf_ref.shape[2]
    w1 = w1_ref[...]
    ys = [jnp.dot(w1, feat_ref[b], preferred_element_type=jnp.float32,
                  precision=_HIGH) for b in range(B)]
    cnt1 = float(B * N)
    mean1 = sum(jnp.sum(y, axis=1, keepdims=True) for y in ys) / cnt1
    var1 = sum(jnp.sum((y - mean1) ** 2, axis=1, keepdims=True)
               for y in ys) / cnt1
    sc1 = g1_ref[...] * jax.lax.rsqrt(var1 + _EPS_BN)
    sh1 = b1_ref[...] - mean1 * sc1
    for b in range(B):
        f_ref[b] = jnp.maximum(ys[b] * sc1 + sh1, 0.0)

    cnt2 = float(B * Mtot)
    w2 = w2_ref[...]
    xmean = sum(jnp.sum(tf_ref[b], axis=1, keepdims=True)
                for b in range(B)) / cnt2
    smom = sum(jax.lax.dot_general(tf_ref[b], tf_ref[b],
                                   (((1,), (1,)), ((), ())),
                                   preferred_element_type=jnp.float32,
                                   precision=_HIGH) for b in range(B))
    mu2 = jnp.dot(w2, xmean, preferred_element_type=jnp.float32,
                  precision=_HIGH)
    ey2 = jnp.sum(jnp.dot(w2, smom, preferred_element_type=jnp.float32,
                          precision=_HIGH) * w2, axis=1,
                  keepdims=True) / cnt2
    var2 = ey2 - mu2 * mu2
    sc2 = g2_ref[...] * jax.lax.rsqrt(var2 + _EPS_BN)
    sc2_ref[...] = sc2
    sh2_ref[...] = b2_ref[...] - mu2 * sc2


def _main_body(xyz_ref, txyz_ref, tf_ref, w2_ref, sc2_ref, sh2_ref,
               pt_ref, gi0_ref, gi1_ref, gi2_ref, w0_ref, w1_ref, w2o_ref):
    S = xyz_ref[0]
    T = txyz_ref[0]
    N = S.shape[0]
    d2 = None
    for c in range(3):
        diff = S[:, c:c + 1] - T[c:c + 1, :]
        d2 = diff * diff if d2 is None else d2 + diff * diff
    iota0 = jax.lax.broadcasted_iota(jnp.int32, d2.shape, 0)
    key = jax.lax.bitcast_convert_type(
        (jax.lax.bitcast_convert_type(d2, jnp.int32) & jnp.int32(~2047))
        | iota0, jnp.float32)
    kmax = jnp.float32(jnp.inf)
    k0 = jnp.min(key, axis=0, keepdims=True)
    m1 = jnp.where(key == k0, kmax, key)
    k1 = jnp.min(m1, axis=0, keepdims=True)
    m2 = jnp.where(m1 == k1, kmax, m1)
    k2 = jnp.min(m2, axis=0, keepdims=True)
    gbase = pl.program_id(0) * N
    recips = []
    for kk, gi_ref in ((k0, gi0_ref), (k1, gi1_ref), (k2, gi2_ref)):
        kbits = jax.lax.bitcast_convert_type(kk, jnp.int32)
        gi_ref[0] = (kbits & jnp.int32(2047)) + gbase
        dq = jax.lax.bitcast_convert_type(kbits & jnp.int32(~2047),
                                          jnp.float32)
        recips.append(1.0 / (jnp.sqrt(dq) + _EPS_D))
    norm = recips[0] + recips[1] + recips[2]
    w0_ref[0] = recips[0] / norm
    w1_ref[0] = recips[1] / norm
    w2o_ref[0] = recips[2] / norm
    y2 = jnp.dot(w2_ref[...], tf_ref[0], preferred_element_type=jnp.float32,
                 precision=_HIGH)
    pt_ref[0] = jnp.maximum(y2 * sc2_ref[...] + sh2_ref[...], 0.0)


def _sc_gather_body(ftab, gi0, gi1, gi2, r0o, r1o, r2o,
                    ix0, ix1, ix2,
                    ga0, ga1, ga2, gb0, gb1, gb2,
                    sa0, sa1, sa2, sb0, sb1, sb2, rpw, nc):
    wid = lax.axis_index("s") * nc + lax.axis_index("c")
    base = wid * rpw
    nch = rpw // SC_CHUNK
    bufs = ((ga0, ga1, ga2, sa0, sa1, sa2),
            (gb0, gb1, gb2, sb0, sb1, sb2))
    pltpu.sync_copy(gi0.at[pl.ds(base, rpw)], ix0)
    pltpu.sync_copy(gi1.at[pl.ds(base, rpw)], ix1)
    pltpu.sync_copy(gi2.at[pl.ds(base, rpw)], ix2)

    def fire(ch, buf):
        r0, r1, r2, s0, s1, s2 = bufs[buf]
        sl = pl.ds(ch * SC_CHUNK, SC_CHUNK)
        return [pltpu.async_copy(ftab.at[ix0.at[sl]], r0, s0),
                pltpu.async_copy(ftab.at[ix1.at[sl]], r1, s1),
                pltpu.async_copy(ftab.at[ix2.at[sl]], r2, s2)]

    def consume(ch, buf, cps):
        for cp in cps:
            cp.wait()
        r0, r1, r2 = bufs[buf][:3]
        gsl = pl.ds(base + ch * SC_CHUNK, SC_CHUNK)
        pltpu.sync_copy(r0, r0o.at[gsl])
        pltpu.sync_copy(r1, r1o.at[gsl])
        pltpu.sync_copy(r2, r2o.at[gsl])

    prev = fire(0, 0)
    for ch in range(1, nch):
        cur = fire(ch, ch % 2)
        consume(ch - 1, (ch - 1) % 2, prev)
        prev = cur
    consume(nch - 1, (nch - 1) % 2, prev)


def _combine_body(pt_ref, r0_ref, r1_ref, r2_ref, w0_ref, w1_ref, w2_ref,
                  o_ref):
    interp_t = (w0_ref[0] * r0_ref[0]
                + w1_ref[0] * r1_ref[0]
                + w2_ref[0] * r2_ref[0])
    o_ref[0] = pt_ref[0] + jnp.transpose(interp_t)


@jax.jit
def kernel(xyz, feature, target_xyz, target_feature, W1, gamma1, beta1, W2,
           gamma2, beta2):
    B, N, _ = xyz.shape
    M = target_xyz.shape[1]
    C2 = W1.shape[0]
    BM = B * M
    txyz_t = jnp.transpose(target_xyz, (0, 2, 1))
    g1 = gamma1.reshape(C2, 1)
    b1 = beta1.reshape(C2, 1)
    g2 = gamma2.reshape(C2, 1)
    b2 = beta2.reshape(C2, 1)

    f, sc2, sh2 = pl.pallas_call(
        _prep_body,
        out_shape=[
            jax.ShapeDtypeStruct((B, C2, N), jnp.float32),
            jax.ShapeDtypeStruct((C2, 1), jnp.float32),
            jax.ShapeDtypeStruct((C2, 1), jnp.float32),
        ],
    )(feature, W1, g1, b1, target_feature, W2, g2, b2)

    grid = (B, M // MBLK)
    blk_row_i = jax.ShapeDtypeStruct((B, 1, M), jnp.int32)
    blk_row_f = jax.ShapeDtypeStruct((B, 1, M), jnp.float32)
    row_spec = pl.BlockSpec((1, 1, MBLK), lambda b, j: (b, 0, j))
    part, gi0, gi1, gi2, w0, w1, w2 = pl.pallas_call(
        _main_body,
        grid=grid,
        in_specs=[
            pl.BlockSpec((1, N, 3), lambda b, j: (b, 0, 0)),
            pl.BlockSpec((1, 3, MBLK), lambda b, j: (b, 0, j)),
            pl.BlockSpec((1, C2, MBLK), lambda b, j: (b, 0, j)),
            pl.BlockSpec((C2, C2), lambda b, j: (0, 0)),
            pl.BlockSpec((C2, 1), lambda b, j: (0, 0)),
            pl.BlockSpec((C2, 1), lambda b, j: (0, 0)),
        ],
        out_specs=[
            pl.BlockSpec((1, C2, MBLK), lambda b, j: (b, 0, j)),
            row_spec, row_spec, row_spec, row_spec, row_spec, row_spec,
        ],
        out_shape=[
            jax.ShapeDtypeStruct((B, C2, M), jnp.float32),
            blk_row_i, blk_row_i, blk_row_i,
            blk_row_f, blk_row_f, blk_row_f,
        ],
    )(xyz, txyz_t, target_feature, W2, sc2, sh2)

    ftab = jnp.transpose(f, (0, 2, 1)).reshape(B * N, C2)
    info = plsc.get_sparse_core_info()
    nw = info.num_cores * info.num_subcores
    rpw = BM // nw
    mesh = plsc.VectorSubcoreMesh(core_axis_name="c", subcore_axis_name="s")
    rows_ty = jax.ShapeDtypeStruct((BM, C2), jnp.float32)
    r0, r1, r2 = pl.kernel(
        functools.partial(_sc_gather_body, rpw=rpw, nc=info.num_cores),
        mesh=mesh,
        out_type=[rows_ty, rows_ty, rows_ty],
        scratch_types=(
            [pltpu.VMEM((rpw,), jnp.int32)] * 3
            + [pltpu.VMEM((SC_CHUNK, C2), jnp.float32)] * 6
            + [pltpu.SemaphoreType.DMA] * 6
        ),
    )(ftab, gi0.reshape(BM), gi1.reshape(BM), gi2.reshape(BM))
    r0, r1, r2 = (r.reshape(B, M, C2) for r in (r0, r1, r2))

    cgrid = (B, M // CBLK)
    blk_spec = pl.BlockSpec((1, CBLK, C2), lambda b, j: (b, j, 0))
    col_spec = pl.BlockSpec((1, CBLK, 1), lambda b, j: (b, j, 0))
    cm_spec = pl.BlockSpec((1, C2, CBLK), lambda b, j: (b, 0, j))
    out = pl.pallas_call(
        _combine_body,
        grid=cgrid,
        in_specs=[cm_spec, blk_spec, blk_spec, blk_spec,
                  col_spec, col_spec, col_spec],
        out_specs=cm_spec,
        out_shape=jax.ShapeDtypeStruct((B, C2, M), jnp.float32),
    )(part, r0, r1, r2,
      jnp.transpose(w0, (0, 2, 1)), jnp.transpose(w1, (0, 2, 1)),
      jnp.transpose(w2, (0, 2, 1)))
    return out


# speedup vs baseline: 1.1923x; 1.0115x over previous
import functools

import jax
import jax.numpy as jnp
from jax import lax
from jax.experimental import pallas as pl
from jax.experimental.pallas import tpu as pltpu
from jax.experimental.pallas import tpu_sc as plsc

_HIGH = jax.lax.Precision.HIGHEST
_EPS_BN = 1e-5
_EPS_D = 1e-8

MBLK = 1024
CBLK = 2048
SC_CHUNK = 128


def _prep_body(feat_ref, w1_ref, g1_ref, b1_ref, tf_ref, w2_ref, g2_ref,
               b2_ref, f_ref, sc2_ref, sh2_ref):
    B = feat_ref.shape[0]
    N = feat_ref.shape[2]
    Mtot = tf_ref.shape[2]
    w1 = w1_ref[...]
    ys = [jnp.dot(w1, feat_ref[b], preferred_element_type=jnp.float32,
                  precision=_HIGH) for b in range(B)]
    cnt1 = float(B * N)
    mean1 = sum(jnp.sum(y, axis=1, keepdims=True) for y in ys) / cnt1
    var1 = sum(jnp.sum((y - mean1) ** 2, axis=1, keepdims=True)
               for y in ys) / cnt1
    sc1 = g1_ref[...] * jax.lax.rsqrt(var1 + _EPS_BN)
    sh1 = b1_ref[...] - mean1 * sc1
    for b in range(B):
        f_ref[b] = jnp.transpose(jnp.maximum(ys[b] * sc1 + sh1, 0.0))

    cnt2 = float(B * Mtot)
    w2 = w2_ref[...]
    xmean = sum(jnp.sum(tf_ref[b], axis=1, keepdims=True)
                for b in range(B)) / cnt2
    smom = sum(jax.lax.dot_general(tf_ref[b], tf_ref[b],
                                   (((1,), (1,)), ((), ())),
                                   preferred_element_type=jnp.float32,
                                   precision=_HIGH) for b in range(B))
    mu2 = jnp.dot(w2, xmean, preferred_element_type=jnp.float32,
                  precision=_HIGH)
    ey2 = jnp.sum(jnp.dot(w2, smom, preferred_element_type=jnp.float32,
                          precision=_HIGH) * w2, axis=1,
                  keepdims=True) / cnt2
    var2 = ey2 - mu2 * mu2
    sc2 = g2_ref[...] * jax.lax.rsqrt(var2 + _EPS_BN)
    sc2_ref[...] = sc2
    sh2_ref[...] = b2_ref[...] - mu2 * sc2


def _main_body(xyz_ref, txyz_ref, tf_ref, w2_ref, sc2_ref, sh2_ref,
               pt_ref, gi0_ref, gi1_ref, gi2_ref, w0_ref, w1_ref, w2o_ref):
    S = xyz_ref[0]
    T = txyz_ref[0]
    N = S.shape[0]
    d2 = None
    for c in range(3):
        diff = S[:, c:c + 1] - T[c:c + 1, :]
        d2 = diff * diff if d2 is None else d2 + diff * diff
    iota0 = jax.lax.broadcasted_iota(jnp.int32, d2.shape, 0)
    key = jax.lax.bitcast_convert_type(
        (jax.lax.bitcast_convert_type(d2, jnp.int32) & jnp.int32(~2047))
        | iota0, jnp.float32)
    kmax = jnp.float32(jnp.inf)
    k0 = jnp.min(key, axis=0, keepdims=True)
    m1 = jnp.where(key == k0, kmax, key)
    k1 = jnp.min(m1, axis=0, keepdims=True)
    m2 = jnp.where(m1 == k1, kmax, m1)
    k2 = jnp.min(m2, axis=0, keepdims=True)
    gbase = pl.program_id(0) * N
    recips = []
    for kk, gi_ref in ((k0, gi0_ref), (k1, gi1_ref), (k2, gi2_ref)):
        kbits = jax.lax.bitcast_convert_type(kk, jnp.int32)
        gi_ref[0] = (kbits & jnp.int32(2047)) + gbase
        dq = jax.lax.bitcast_convert_type(kbits & jnp.int32(~2047),
                                          jnp.float32)
        recips.append(1.0 / (jnp.sqrt(dq) + _EPS_D))
    norm = recips[0] + recips[1] + recips[2]
    w0_ref[0] = recips[0] / norm
    w1_ref[0] = recips[1] / norm
    w2o_ref[0] = recips[2] / norm
    y2 = jnp.dot(w2_ref[...], tf_ref[0], preferred_element_type=jnp.float32,
                 precision=_HIGH)
    pt_ref[0] = jnp.maximum(y2 * sc2_ref[...] + sh2_ref[...], 0.0)


def _sc_gather_body(ftab, gi0, gi1, gi2, r0o, r1o, r2o,
                    ix0, ix1, ix2,
                    ga0, ga1, ga2, gb0, gb1, gb2,
                    sa0, sa1, sa2, sb0, sb1, sb2, rpw, nc):
    wid = lax.axis_index("s") * nc + lax.axis_index("c")
    base = wid * rpw
    nch = rpw // SC_CHUNK
    bufs = ((ga0, ga1, ga2, sa0, sa1, sa2),
            (gb0, gb1, gb2, sb0, sb1, sb2))
    pltpu.sync_copy(gi0.at[pl.ds(base, rpw)], ix0)
    pltpu.sync_copy(gi1.at[pl.ds(base, rpw)], ix1)
    pltpu.sync_copy(gi2.at[pl.ds(base, rpw)], ix2)

    def fire(ch, buf):
        r0, r1, r2, s0, s1, s2 = bufs[buf]
        sl = pl.ds(ch * SC_CHUNK, SC_CHUNK)
        return [pltpu.async_copy(ftab.at[ix0.at[sl]], r0, s0),
                pltpu.async_copy(ftab.at[ix1.at[sl]], r1, s1),
                pltpu.async_copy(ftab.at[ix2.at[sl]], r2, s2)]

    def consume(ch, buf, cps):
        for cp in cps:
            cp.wait()
        r0, r1, r2 = bufs[buf][:3]
        gsl = pl.ds(base + ch * SC_CHUNK, SC_CHUNK)
        pltpu.sync_copy(r0, r0o.at[gsl])
        pltpu.sync_copy(r1, r1o.at[gsl])
        pltpu.sync_copy(r2, r2o.at[gsl])

    prev = fire(0, 0)
    for ch in range(1, nch):
        cur = fire(ch, ch % 2)
        consume(ch - 1, (ch - 1) % 2, prev)
        prev = cur
    consume(nch - 1, (nch - 1) % 2, prev)


def _combine_body(pt_ref, r0_ref, r1_ref, r2_ref, w0_ref, w1_ref, w2_ref,
                  o_ref):
    interp_t = (w0_ref[0] * r0_ref[0]
                + w1_ref[0] * r1_ref[0]
                + w2_ref[0] * r2_ref[0])
    o_ref[0] = pt_ref[0] + jnp.transpose(interp_t)


@jax.jit
def kernel(xyz, feature, target_xyz, target_feature, W1, gamma1, beta1, W2,
           gamma2, beta2):
    B, N, _ = xyz.shape
    M = target_xyz.shape[1]
    C2 = W1.shape[0]
    BM = B * M
    txyz_t = jnp.transpose(target_xyz, (0, 2, 1))
    g1 = gamma1.reshape(C2, 1)
    b1 = beta1.reshape(C2, 1)
    g2 = gamma2.reshape(C2, 1)
    b2 = beta2.reshape(C2, 1)

    f, sc2, sh2 = pl.pallas_call(
        _prep_body,
        out_shape=[
            jax.ShapeDtypeStruct((B, N, C2), jnp.float32),
            jax.ShapeDtypeStruct((C2, 1), jnp.float32),
            jax.ShapeDtypeStruct((C2, 1), jnp.float32),
        ],
    )(feature, W1, g1, b1, target_feature, W2, g2, b2)

    grid = (B, M // MBLK)
    blk_row_i = jax.ShapeDtypeStruct((B, 1, M), jnp.int32)
    blk_row_f = jax.ShapeDtypeStruct((B, 1, M), jnp.float32)
    row_spec = pl.BlockSpec((1, 1, MBLK), lambda b, j: (b, 0, j))
    part, gi0, gi1, gi2, w0, w1, w2 = pl.pallas_call(
        _main_body,
        grid=grid,
        in_specs=[
            pl.BlockSpec((1, N, 3), lambda b, j: (b, 0, 0)),
            pl.BlockSpec((1, 3, MBLK), lambda b, j: (b, 0, j)),
            pl.BlockSpec((1, C2, MBLK), lambda b, j: (b, 0, j)),
            pl.BlockSpec((C2, C2), lambda b, j: (0, 0)),
            pl.BlockSpec((C2, 1), lambda b, j: (0, 0)),
            pl.BlockSpec((C2, 1), lambda b, j: (0, 0)),
        ],
        out_specs=[
            pl.BlockSpec((1, C2, MBLK), lambda b, j: (b, 0, j)),
            row_spec, row_spec, row_spec, row_spec, row_spec, row_spec,
        ],
        out_shape=[
            jax.ShapeDtypeStruct((B, C2, M), jnp.float32),
            blk_row_i, blk_row_i, blk_row_i,
            blk_row_f, blk_row_f, blk_row_f,
        ],
    )(xyz, txyz_t, target_feature, W2, sc2, sh2)

    ftab = f.reshape(B * N, C2)
    info = plsc.get_sparse_core_info()
    nw = info.num_cores * info.num_subcores
    rpw = BM // nw
    mesh = plsc.VectorSubcoreMesh(core_axis_name="c", subcore_axis_name="s")
    rows_ty = jax.ShapeDtypeStruct((BM, C2), jnp.float32)
    r0, r1, r2 = pl.kernel(
        functools.partial(_sc_gather_body, rpw=rpw, nc=info.num_cores),
        mesh=mesh,
        out_type=[rows_ty, rows_ty, rows_ty],
        scratch_types=(
            [pltpu.VMEM((rpw,), jnp.int32)] * 3
            + [pltpu.VMEM((SC_CHUNK, C2), jnp.float32)] * 6
            + [pltpu.SemaphoreType.DMA] * 6
        ),
    )(ftab, gi0.reshape(BM), gi1.reshape(BM), gi2.reshape(BM))
    r0, r1, r2 = (r.reshape(B, M, C2) for r in (r0, r1, r2))

    cgrid = (B, M // CBLK)
    blk_spec = pl.BlockSpec((1, CBLK, C2), lambda b, j: (b, j, 0))
    col_spec = pl.BlockSpec((1, CBLK, 1), lambda b, j: (b, j, 0))
    cm_spec = pl.BlockSpec((1, C2, CBLK), lambda b, j: (b, 0, j))
    out = pl.pallas_call(
        _combine_body,
        grid=cgrid,
        in_specs=[cm_spec, blk_spec, blk_spec, blk_spec,
                  col_spec, col_spec, col_spec],
        out_specs=cm_spec,
        out_shape=jax.ShapeDtypeStruct((B, C2, M), jnp.float32),
    )(part, r0, r1, r2,
      jnp.transpose(w0, (0, 2, 1)), jnp.transpose(w1, (0, 2, 1)),
      jnp.transpose(w2, (0, 2, 1)))
    return out
